# baseline (speedup 1.0000x reference)
import functools

import jax
import jax.numpy as jnp
from jax import lax
from jax.experimental import pallas as pl
from jax.experimental.pallas import tpu as pltpu
from jax.experimental.pallas import tpu_sc as plsc

N = 320000
D = 128
V = 1024
H = D // 2


def _proj_body(emb_ref, w_ref, b_ref, out_ref):
    out_ref[:] = lax.dot_general(
        emb_ref[:], w_ref[:],
        dimension_numbers=(((1,), (1,)), ((), ())),
        preferred_element_type=jnp.float32,
    ) + b_ref[:]


def _project(emb, W, b):
    return pl.pallas_call(
        _proj_body,
        out_shape=jax.ShapeDtypeStruct((V, D), jnp.float32),
    )(emb, W, b.reshape(1, D))


def _pack_table(P):
    lo = lax.bitcast_convert_type(P[:, :H].astype(jnp.bfloat16), jnp.uint16)
    hi = lax.bitcast_convert_type(P[:, H:].astype(jnp.bfloat16), jnp.uint16)
    packed = lo.astype(jnp.uint32) | (hi.astype(jnp.uint32) << 16)
    return lax.bitcast_convert_type(packed, jnp.int32).reshape(V * H)


_info = plsc.get_sparse_core_info()
_NC, _NS = _info.num_cores, _info.num_subcores
_NW = _NC * _NS
N_SC = 196608
ROWS_W = N_SC // _NW
CF = 128
NF = ROWS_W // CF
NBUF = 3
TB = 512
N_TC = N - N_SC
NB = N_TC // TB

_mesh = plsc.VectorSubcoreMesh(core_axis_name="c", subcore_axis_name="s")


@functools.partial(
    pl.kernel,
    mesh=_mesh,
    out_type=jax.ShapeDtypeStruct((N_SC, D), jnp.float32),
    compiler_params=pltpu.CompilerParams(needs_layout_passes=False),
    scratch_types=[
        pltpu.VMEM((ROWS_W,), jnp.int32),
        pltpu.VMEM((CF, D), jnp.float32),
        pltpu.VMEM((CF, D), jnp.float32),
        pltpu.VMEM((CF, D), jnp.float32),
        pltpu.VMEM((V * H,), jnp.int32),
        pltpu.SemaphoreType.DMA,
        pltpu.SemaphoreType.DMA,
        pltpu.SemaphoreType.DMA,
        pltpu.SemaphoreType.DMA,
        pltpu.SemaphoreType.DMA,
        pltpu.SemaphoreType.DMA,
    ],
)
def _sc_add(p_hbm, x_hbm, t_hbm, out_hbm,
            tslab, x0, x1, x2, p_tile,
            sx0, sx1, sx2, so0, so1, so2):
    x = (x0, x1, x2)
    sx = (sx0, sx1, sx2)
    so = (so0, so1, so2)

    wid = lax.axis_index("s") * _NC + lax.axis_index("c")
    base = wid * ROWS_W

    pltpu.sync_copy(t_hbm.at[pl.ds(base, ROWS_W)], tslab)
    pltpu.sync_copy(p_hbm, p_tile)

    def fire_in(c, b):
        pltpu.async_copy(x_hbm.at[pl.ds(base + c * CF, CF), :], x[b], sx[b])

    def wait_x(c, b):
        pltpu.make_async_copy(
            x_hbm.at[pl.ds(base + c * CF, CF), :], x[b], sx[b]).wait()

    def wait_out(c, b):
        pltpu.make_async_copy(
            x[b], out_hbm.at[pl.ds(base + c * CF, CF), :], so[b]).wait()

    cols = [lax.iota(jnp.int32, 16) + 16 * j for j in range(H // 16)]
    zvec = jnp.minimum(tslab[pl.ds(0, 16)], 0)

    def accumulate(off, xb, rows):
        def body(g, carry):
            gvec = jnp.broadcast_to(off + g * 16, (16,)) + zvec
            for half in range(2):
                tbs = [plsc.load_gather(tslab, [gvec + (half * 8 + k)]) * H
                       for k in range(8)]
                for j in range(H // 16):
                    pvs = [plsc.load_gather(p_tile, [tbs[k] + cols[j]])
                           for k in range(8)]
                    for k in range(8):
                        r = g * 16 + half * 8 + k
                        lo = plsc.bitcast(pvs[k] << 16, jnp.float32)
                        hi = plsc.bitcast(
                            pvs[k] & jnp.int32(-65536), jnp.float32)
                        plsc.addupdate(xb.at[r, pl.ds(16 * j, 16)], lo)
                        plsc.addupdate(xb.at[r, pl.ds(H + 16 * j, 16)], hi)
            return carry
        lax.fori_loop(0, rows // 16, body, 0)

    fire_in(0, 0)

    def step(k, carry):
        for bb in range(NBUF):
            c = k * NBUF + bb
            b = bb
            bn = (bb + 1) % NBUF

            @pl.when(c >= 2)
            def _():
                wait_out(c - 2, bn)

            @pl.when(c + 1 < NF)
            def _():
                fire_in(c + 1, bn)

            wait_x(c, b)
            accumulate(c * CF, x[b], CF)
            pltpu.async_copy(
                x[b], out_hbm.at[pl.ds(base + c * CF, CF), :], so[b])
        return carry

    lax.fori_loop(0, NF // NBUF, step, 0)

    wait_out(NF - 2, (NF - 2) % NBUF)
    wait_out(NF - 1, (NF - 1) % NBUF)


def _tc_body(t_ref, x_ref, p_ref, o_ref):
    oh = (lax.broadcasted_iota(jnp.int32, (V, TB), 0)
          == t_ref[0]).astype(jnp.bfloat16)
    o_ref[:] = x_ref[:] + lax.dot_general(
        oh, p_ref[:],
        dimension_numbers=(((0,), (0,)), ((), ())),
        preferred_element_type=jnp.float32)


def _tc_add(t3, x_tc, Pb):
    return pl.pallas_call(
        _tc_body,
        grid=(NB,),
        in_specs=[
            pl.BlockSpec((1, 1, TB), lambda i: (i, 0, 0)),
            pl.BlockSpec((TB, D), lambda i: (i, 0)),
            pl.BlockSpec((V, D), lambda i: (0, 0)),
        ],
        out_specs=pl.BlockSpec((TB, D), lambda i: (i, 0)),
        out_shape=jax.ShapeDtypeStruct((N_TC, D), jnp.float32),
    )(t3, x_tc, Pb)


def kernel(x, t, emb, W, b):
    P = _project(emb, W, b)
    sc_out = _sc_add(_pack_table(P), x[:N_SC], t[:N_SC])
    t3 = t[N_SC:].reshape(NB, 1, TB)
    tc_out = _tc_add(t3, x[N_SC:], P.astype(jnp.bfloat16))
    return jnp.concatenate([sc_out, tc_out], axis=0)


# speedup vs baseline: 3.2180x; 3.2180x over previous
import functools

import jax
import jax.numpy as jnp
from jax import lax
from jax.experimental import pallas as pl
from jax.experimental.pallas import tpu as pltpu
from jax.experimental.pallas import tpu_sc as plsc

N = 320000
D = 128
V = 1024
H = D // 2


def _proj_body(emb_ref, w_ref, b_ref, out_ref):
    out_ref[:] = lax.dot_general(
        emb_ref[:], w_ref[:],
        dimension_numbers=(((1,), (1,)), ((), ())),
        preferred_element_type=jnp.float32,
    ) + b_ref[:]


def _project(emb, W, b):
    return pl.pallas_call(
        _proj_body,
        out_shape=jax.ShapeDtypeStruct((V, D), jnp.float32),
    )(emb, W, b.reshape(1, D))


def _pack_table(P):
    lo = lax.bitcast_convert_type(P[:, :H].astype(jnp.bfloat16), jnp.uint16)
    hi = lax.bitcast_convert_type(P[:, H:].astype(jnp.bfloat16), jnp.uint16)
    packed = lo.astype(jnp.uint32) | (hi.astype(jnp.uint32) << 16)
    return lax.bitcast_convert_type(packed, jnp.int32).reshape(V * H)


_info = plsc.get_sparse_core_info()
_NC, _NS = _info.num_cores, _info.num_subcores
_NW = _NC * _NS
ROWS_W = N // _NW
CF = 128
NF = ROWS_W // CF
TAIL = ROWS_W - NF * CF
NBUF = 3

_mesh = plsc.VectorSubcoreMesh(core_axis_name="c", subcore_axis_name="s")


@functools.partial(
    pl.kernel,
    mesh=_mesh,
    out_type=jax.ShapeDtypeStruct((N, D), jnp.float32),
    compiler_params=pltpu.CompilerParams(needs_layout_passes=False),
    scratch_types=[
        pltpu.VMEM((ROWS_W,), jnp.int32),
        pltpu.VMEM((CF, D), jnp.float32),
        pltpu.VMEM((CF, D), jnp.float32),
        pltpu.VMEM((CF, D), jnp.float32),
        pltpu.VMEM((TAIL, D), jnp.float32),
        pltpu.VMEM((V * H,), jnp.int32),
        pltpu.SemaphoreType.DMA,
        pltpu.SemaphoreType.DMA,
        pltpu.SemaphoreType.DMA,
        pltpu.SemaphoreType.DMA,
        pltpu.SemaphoreType.DMA,
        pltpu.SemaphoreType.DMA,
    ],
)
def _sc_add(p_hbm, x_hbm, t_hbm, out_hbm,
            tslab, x0, x1, x2, xt, p_tile,
            sx0, sx1, sx2, so0, so1, so2):
    x = (x0, x1, x2)
    sx = (sx0, sx1, sx2)
    so = (so0, so1, so2)

    wid = lax.axis_index("s") * _NC + lax.axis_index("c")
    base = wid * ROWS_W

    pltpu.sync_copy(t_hbm.at[pl.ds(base, ROWS_W)], tslab)
    pltpu.sync_copy(p_hbm, p_tile)

    def fire_in(c, b):
        pltpu.async_copy(x_hbm.at[pl.ds(base + c * CF, CF), :], x[b], sx[b])

    def wait_x(c, b):
        pltpu.make_async_copy(
            x_hbm.at[pl.ds(base + c * CF, CF), :], x[b], sx[b]).wait()

    def wait_out(c, b):
        pltpu.make_async_copy(
            x[b], out_hbm.at[pl.ds(base + c * CF, CF), :], so[b]).wait()

    cols = [lax.iota(jnp.int32, 16) + 16 * j for j in range(H // 16)]
    zvec = jnp.minimum(tslab[pl.ds(0, 16)], 0)

    def accumulate(off, xb, rows):
        def body(g, carry):
            gvec = jnp.broadcast_to(off + g * 16, (16,)) + zvec
            for half in range(2):
                tbs = [plsc.load_gather(tslab, [gvec + (half * 8 + k)]) * H
                       for k in range(8)]
                for j in range(H // 16):
                    pvs = [plsc.load_gather(p_tile, [tbs[k] + cols[j]])
                           for k in range(8)]
                    for k in range(8):
                        r = g * 16 + half * 8 + k
                        lo = plsc.bitcast(pvs[k] << 16, jnp.float32)
                        hi = plsc.bitcast(
                            pvs[k] & jnp.int32(-65536), jnp.float32)
                        plsc.addupdate(xb.at[r, pl.ds(16 * j, 16)], lo)
                        plsc.addupdate(xb.at[r, pl.ds(H + 16 * j, 16)], hi)
            return carry
        lax.fori_loop(0, rows // 16, body, 0)

    fire_in(0, 0)

    def step(k, carry):
        for bb in range(NBUF):
            c = k * NBUF + bb
            b = bb
            bn = (bb + 1) % NBUF

            @pl.when(c >= 2)
            def _():
                wait_out(c - 2, bn)

            @pl.when(c + 1 < NF)
            def _():
                fire_in(c + 1, bn)

            wait_x(c, b)
            accumulate(c * CF, x[b], CF)
            pltpu.async_copy(
                x[b], out_hbm.at[pl.ds(base + c * CF, CF), :], so[b])
        return carry

    lax.fori_loop(0, NF // NBUF, step, 0)

    wait_out(NF - 2, (NF - 2) % NBUF)
    wait_out(NF - 1, (NF - 1) % NBUF)

    toff = base + NF * CF
    pltpu.sync_copy(x_hbm.at[pl.ds(toff, TAIL), :], xt)
    accumulate(NF * CF, xt, TAIL)
    pltpu.sync_copy(xt, out_hbm.at[pl.ds(toff, TAIL), :])


def kernel(x, t, emb, W, b):
    P = _project(emb, W, b)
    return _sc_add(_pack_table(P), x, t)


# speedup vs baseline: 3.2527x; 1.0108x over previous
import functools

import jax
import jax.numpy as jnp
from jax import lax
from jax.experimental import pallas as pl
from jax.experimental.pallas import tpu as pltpu
from jax.experimental.pallas import tpu_sc as plsc

N = 320000
D = 128
V = 1024
H = D // 2


def _proj_body(emb_ref, w_ref, b_ref, out_ref):
    out_ref[:] = lax.dot_general(
        emb_ref[:], w_ref[:],
        dimension_numbers=(((1,), (1,)), ((), ())),
        preferred_element_type=jnp.float32,
    ) + b_ref[:]


def _project(emb, W, b):
    return pl.pallas_call(
        _proj_body,
        out_shape=jax.ShapeDtypeStruct((V, D), jnp.float32),
    )(emb, W, b.reshape(1, D))


def _pack_table(P):
    lo = lax.bitcast_convert_type(P[:, :H].astype(jnp.bfloat16), jnp.uint16)
    hi = lax.bitcast_convert_type(P[:, H:].astype(jnp.bfloat16), jnp.uint16)
    packed = lo.astype(jnp.uint32) | (hi.astype(jnp.uint32) << 16)
    return lax.bitcast_convert_type(packed, jnp.int32).reshape(V * H)


_info = plsc.get_sparse_core_info()
_NC, _NS = _info.num_cores, _info.num_subcores
_NW = _NC * _NS
ROWS_W = N // _NW
CF = 128
NF = ROWS_W // CF
TAIL = ROWS_W - NF * CF
NBUF = 3

_mesh = plsc.VectorSubcoreMesh(core_axis_name="c", subcore_axis_name="s")


@functools.partial(
    pl.kernel,
    mesh=_mesh,
    out_type=jax.ShapeDtypeStruct((N, D), jnp.float32),
    compiler_params=pltpu.CompilerParams(needs_layout_passes=False),
    scratch_types=[
        pltpu.VMEM((ROWS_W,), jnp.int32),
        pltpu.VMEM((CF, D), jnp.float32),
        pltpu.VMEM((CF, D), jnp.float32),
        pltpu.VMEM((CF, D), jnp.float32),
        pltpu.VMEM((TAIL, D), jnp.float32),
        pltpu.VMEM((V * H,), jnp.int32),
        pltpu.SemaphoreType.DMA,
        pltpu.SemaphoreType.DMA,
        pltpu.SemaphoreType.DMA,
        pltpu.SemaphoreType.DMA,
        pltpu.SemaphoreType.DMA,
        pltpu.SemaphoreType.DMA,
    ],
)
def _sc_add(p_hbm, x_hbm, t_hbm, out_hbm,
            tslab, x0, x1, x2, xt, p_tile,
            sx0, sx1, sx2, so0, so1, so2):
    x = (x0, x1, x2)
    sx = (sx0, sx1, sx2)
    so = (so0, so1, so2)

    wid = lax.axis_index("s") * _NC + lax.axis_index("c")
    base = wid * ROWS_W

    def fire_in(c, b):
        pltpu.async_copy(x_hbm.at[pl.ds(base + c * CF, CF), :], x[b], sx[b])

    def wait_x(c, b):
        pltpu.make_async_copy(
            x_hbm.at[pl.ds(base + c * CF, CF), :], x[b], sx[b]).wait()

    def wait_out(c, b):
        pltpu.make_async_copy(
            x[b], out_hbm.at[pl.ds(base + c * CF, CF), :], so[b]).wait()

    cols = [lax.iota(jnp.int32, 16) + 16 * j for j in range(H // 16)]

    tcp = pltpu.async_copy(t_hbm.at[pl.ds(base, ROWS_W)], tslab, so[0])
    pcp = pltpu.async_copy(p_hbm, p_tile, so[1])
    fire_in(0, 0)
    fire_in(1, 1)
    tcp.wait()
    pcp.wait()

    zvec = jnp.minimum(tslab[pl.ds(0, 16)], 0)

    def accumulate(off, xb, rows):
        def body(g, carry):
            gvec = jnp.broadcast_to(off + g * 16, (16,)) + zvec
            for half in range(2):
                tbs = [plsc.load_gather(tslab, [gvec + (half * 8 + k)]) * H
                       for k in range(8)]
                for j in range(H // 16):
                    pvs = [plsc.load_gather(p_tile, [tbs[k] + cols[j]])
                           for k in range(8)]
                    for k in range(8):
                        r = g * 16 + half * 8 + k
                        lo = plsc.bitcast(pvs[k] << 16, jnp.float32)
                        hi = plsc.bitcast(
                            pvs[k] & jnp.int32(-65536), jnp.float32)
                        plsc.addupdate(xb.at[r, pl.ds(16 * j, 16)], lo)
                        plsc.addupdate(xb.at[r, pl.ds(H + 16 * j, 16)], hi)
            return carry
        lax.fori_loop(0, rows // 16, body, 0)

    def step(k, carry):
        for bb in range(NBUF):
            c = k * NBUF + bb
            b = bb
            bn = (bb + 1) % NBUF

            @pl.when(c >= 2)
            def _():
                wait_out(c - 2, bn)

            @pl.when((c >= 1) & (c + 1 < NF))
            def _():
                fire_in(c + 1, bn)

            wait_x(c, b)
            accumulate(c * CF, x[b], CF)
            pltpu.async_copy(
                x[b], out_hbm.at[pl.ds(base + c * CF, CF), :], so[b])
        return carry

    lax.fori_loop(0, NF // NBUF, step, 0)

    wait_out(NF - 2, (NF - 2) % NBUF)
    wait_out(NF - 1, (NF - 1) % NBUF)

    toff = base + NF * CF
    pltpu.sync_copy(x_hbm.at[pl.ds(toff, TAIL), :], xt)
    accumulate(NF * CF, xt, TAIL)
    pltpu.sync_copy(xt, out_hbm.at[pl.ds(toff, TAIL), :])


def kernel(x, t, emb, W, b):
    P = _project(emb, W, b)
    return _sc_add(_pack_table(P), x, t)
